# baseline (speedup 1.0000x reference)
import functools

import jax
import jax.numpy as jnp
from jax import lax
from jax.experimental import pallas as pl
from jax.experimental.pallas import tpu as pltpu
from jax.experimental.pallas import tpu_sc as plsc

EMBED = 64
CHUNK = 128
NVREG = CHUNK // 16


@functools.lru_cache(maxsize=None)
def _make_gather(n_total: int, vocab: int):
    info = plsc.get_sparse_core_info()
    nc, ns = info.num_cores, info.num_subcores
    nw = nc * ns
    assert n_total % (nw * CHUNK) == 0
    per_w = n_total // nw
    n_chunks = per_w // CHUNK

    mesh = plsc.VectorSubcoreMesh(core_axis_name="c", subcore_axis_name="s")

    @functools.partial(
        pl.kernel,
        mesh=mesh,
        compiler_params=pltpu.CompilerParams(
            use_tc_tiling_on_sc=False, needs_layout_passes=False),
        out_type=jax.ShapeDtypeStruct((n_total, EMBED), jnp.float32),
        scratch_types=[
            pltpu.VMEM((n_chunks, CHUNK), jnp.int32),
            pltpu.VMEM((1, CHUNK), jnp.int32),
            pltpu.VMEM((CHUNK, 2 * EMBED), jnp.float32),
            pltpu.VMEM((CHUNK, EMBED), jnp.float32),
            pltpu.VMEM((NVREG, 16), jnp.int32),
            pltpu.SemaphoreType.DMA,
        ],
    )
    def gather_kernel(ids_hbm, table2_hbm, out_hbm, idx_v, hidx_v, f_v, o_v,
                      gb_v, sem):
        wid = lax.axis_index("s") * nc + lax.axis_index("c")
        pltpu.sync_copy(ids_hbm.at[wid], idx_v)
        base = wid * per_w
        lane = lax.iota(jnp.int32, 16)

        def chunk_body(c, carry):
            for p in range(NVREG):
                v = idx_v[c, pl.ds(p * 16, 16)]
                hidx_v[0, pl.ds(p * 16, 16)] = lax.shift_right_logical(v, 1)
                gb_v[p, :] = lax.shift_left(jnp.bitwise_and(v, 1), 6)
            pltpu.async_copy(table2_hbm.at[hidx_v.at[0]], f_v, sem).wait()
            def col_body(k, carry2):
                for p in range(NVREG):
                    rows = p * 16 + lane
                    cols = gb_v[p, :] + k
                    x = plsc.load_gather(f_v, [rows, cols])
                    plsc.store_scatter(o_v, [rows, jnp.zeros((16,), jnp.int32) + k], x)
                return carry2
            lax.fori_loop(0, EMBED, col_body, 0)
            pltpu.sync_copy(o_v, out_hbm.at[pl.ds(base + c * CHUNK, CHUNK)])
            return carry

        lax.fori_loop(0, n_chunks, chunk_body, 0)

    return gather_kernel


def kernel(input_ids, embedding_table):
    b, h = input_ids.shape
    n = b * h
    v = embedding_table.shape[0]
    info = plsc.get_sparse_core_info()
    nw = info.num_cores * info.num_subcores
    per_w = n // nw
    ids3 = input_ids.reshape(nw, per_w // CHUNK, CHUNK).astype(jnp.int32)
    table2 = jnp.concatenate(
        [embedding_table[0::2], embedding_table[1::2]], axis=1)
    out = _make_gather(n, v)(ids3, table2)
    return out.reshape(b, h, EMBED), embedding_table


# speedup vs baseline: 10.3762x; 10.3762x over previous
import functools

import jax
import jax.numpy as jnp
from jax import lax
from jax.experimental import pallas as pl
from jax.experimental.pallas import tpu as pltpu
from jax.experimental.pallas import tpu_sc as plsc

EMBED = 64
CHUNK = 128


@functools.lru_cache(maxsize=None)
def _make_gather(n_total: int, vocab: int):
    info = plsc.get_sparse_core_info()
    nc, ns = info.num_cores, info.num_subcores
    nw = nc * ns
    assert n_total % (nw * CHUNK) == 0
    per_w = n_total // nw
    n_chunks = per_w // CHUNK

    mesh = plsc.VectorSubcoreMesh(core_axis_name="c", subcore_axis_name="s")

    @functools.partial(
        pl.kernel,
        mesh=mesh,
        compiler_params=pltpu.CompilerParams(
            use_tc_tiling_on_sc=False, needs_layout_passes=False),
        out_type=jax.ShapeDtypeStruct((n_total, EMBED), jnp.float32),
        scratch_types=[
            pltpu.VMEM((n_chunks, CHUNK), jnp.int32),
            pltpu.VMEM((CHUNK, 2 * EMBED), jnp.float32),
            pltpu.SemaphoreType.DMA,
        ],
    )
    def gather_kernel(ids_hbm, tpad_hbm, out_hbm, idx_v, rows_v, sem):
        wid = lax.axis_index("s") * nc + lax.axis_index("c")
        pltpu.sync_copy(ids_hbm.at[wid], idx_v)
        base = wid * per_w

        def chunk_body(c, carry):
            pltpu.async_copy(tpad_hbm.at[idx_v.at[c]], rows_v, sem).wait()
            pltpu.sync_copy(rows_v.at[:, pl.ds(0, EMBED)],
                            out_hbm.at[pl.ds(base + c * CHUNK, CHUNK)])
            return carry

        lax.fori_loop(0, n_chunks, chunk_body, 0)

    return gather_kernel


def kernel(input_ids, embedding_table):
    b, h = input_ids.shape
    n = b * h
    v = embedding_table.shape[0]
    info = plsc.get_sparse_core_info()
    nw = info.num_cores * info.num_subcores
    per_w = n // nw
    ids3 = input_ids.reshape(nw, per_w // CHUNK, CHUNK).astype(jnp.int32)
    tpad = jnp.pad(embedding_table, ((0, 0), (0, 2 * EMBED - EMBED)))
    out = _make_gather(n, v)(ids3, tpad)
    return out.reshape(b, h, EMBED), embedding_table
